# baseline (speedup 1.0000x reference)
import functools

import jax
import jax.numpy as jnp
from jax.experimental import pallas as pl
from jax.experimental.pallas import tpu as pltpu

_B, _N, _F, _E = 1024, 500, 64, 16
_TB = 32
_ROWS = _TB * _N


def _fused_body(x_ref, w_ref, b_ref, wvt_ref, g_ref, p_ref, bv_ref,
                act_ref, val_ref):
    y = jnp.dot(x_ref[...], w_ref[...], preferred_element_type=jnp.float32)
    z = jnp.maximum(y + b_ref[...], 0.0) * wvt_ref[...]
    vg = jnp.dot(g_ref[...], z, preferred_element_type=jnp.float32)
    val_ref[...] = jnp.sum(vg, axis=1, keepdims=True) + bv_ref[0, 0]

    a0 = jax.nn.sigmoid(p_ref[0, 0])
    a1 = jax.nn.sigmoid(p_ref[0, 1]) * 0.5
    col = jax.lax.broadcasted_iota(jnp.int32, (_TB, 2 * _N), 1)
    act_ref[...] = jnp.where(col % 2 == 0, a0, a1)


@functools.partial(jax.jit, static_argnums=())
def kernel(node_features_gen, W_embed, b_embed, param, W_val, b_val):
    x2 = node_features_gen.reshape(_B * _N, _F)
    wv = W_val.reshape(_N, _E)
    wvt = jnp.tile(wv, (_TB, 1))
    r_idx = jax.lax.broadcasted_iota(jnp.int32, (_TB, _ROWS), 1)
    g_idx = jax.lax.broadcasted_iota(jnp.int32, (_TB, _ROWS), 0)
    gmat = (r_idx // _N == g_idx).astype(jnp.float32)

    grid = (_B // _TB,)
    acts, val = pl.pallas_call(
        _fused_body,
        grid=grid,
        in_specs=[
            pl.BlockSpec((_ROWS, _F), lambda i: (i, 0)),
            pl.BlockSpec((_F, _E), lambda i: (0, 0)),
            pl.BlockSpec((1, _E), lambda i: (0, 0)),
            pl.BlockSpec((_ROWS, _E), lambda i: (0, 0)),
            pl.BlockSpec((_TB, _ROWS), lambda i: (0, 0)),
            pl.BlockSpec((1, 2), lambda i: (0, 0)),
            pl.BlockSpec((1, 1), lambda i: (0, 0)),
        ],
        out_specs=[
            pl.BlockSpec((_TB, 2 * _N), lambda i: (i, 0)),
            pl.BlockSpec((_TB, 1), lambda i: (i, 0)),
        ],
        out_shape=[
            jax.ShapeDtypeStruct((_B, 2 * _N), jnp.float32),
            jax.ShapeDtypeStruct((_B, 1), jnp.float32),
        ],
    )(x2, W_embed, b_embed.reshape(1, _E), wvt, gmat,
      param.reshape(1, 2), b_val.reshape(1, 1))
    return acts, val.reshape(_B)


# speedup vs baseline: 1.4231x; 1.4231x over previous
import functools

import jax
import jax.numpy as jnp
from jax.experimental import pallas as pl
from jax.experimental.pallas import tpu as pltpu

_B, _N, _F, _E = 1024, 500, 64, 16
_TB = 32
_ROWS = _TB * _N


def _fused_body(x_ref, w_ref, b_ref, wvt_ref, g_ref, p_ref, bv_ref,
                act_ref, val_ref):
    x = x_ref[...].reshape(_ROWS, _F)
    y = jnp.dot(x, w_ref[...], preferred_element_type=jnp.float32)
    z = jnp.maximum(y + b_ref[...], 0.0) * wvt_ref[...]
    vg = jnp.dot(g_ref[...], z, preferred_element_type=jnp.float32)
    val_ref[...] = jnp.sum(vg, axis=1, keepdims=True) + bv_ref[0, 0]

    a0 = jax.nn.sigmoid(p_ref[0, 0])
    a1 = jax.nn.sigmoid(p_ref[0, 1]) * 0.5
    col = jax.lax.broadcasted_iota(jnp.int32, (_TB, 2 * _N), 1)
    act_ref[...] = jnp.where(col % 2 == 0, a0, a1)


@functools.partial(jax.jit, static_argnums=())
def kernel(node_features_gen, W_embed, b_embed, param, W_val, b_val):
    wv = W_val.reshape(_N, _E)
    wvt = jnp.tile(wv, (_TB, 1))
    r_idx = jax.lax.broadcasted_iota(jnp.int32, (_TB, _ROWS), 1)
    g_idx = jax.lax.broadcasted_iota(jnp.int32, (_TB, _ROWS), 0)
    gmat = (r_idx // _N == g_idx).astype(jnp.float32)

    grid = (_B // _TB,)
    acts, val = pl.pallas_call(
        _fused_body,
        grid=grid,
        in_specs=[
            pl.BlockSpec((_TB, _N, _F), lambda i: (i, 0, 0)),
            pl.BlockSpec((_F, _E), lambda i: (0, 0)),
            pl.BlockSpec((1, _E), lambda i: (0, 0)),
            pl.BlockSpec((_ROWS, _E), lambda i: (0, 0)),
            pl.BlockSpec((_TB, _ROWS), lambda i: (0, 0)),
            pl.BlockSpec((1, 2), lambda i: (0, 0)),
            pl.BlockSpec((1, 1), lambda i: (0, 0)),
        ],
        out_specs=[
            pl.BlockSpec((_TB, 2 * _N), lambda i: (i, 0)),
            pl.BlockSpec((_TB, 1), lambda i: (i, 0)),
        ],
        out_shape=[
            jax.ShapeDtypeStruct((_B, 2 * _N), jnp.float32),
            jax.ShapeDtypeStruct((_B, 1), jnp.float32),
        ],
    )(node_features_gen, W_embed, b_embed.reshape(1, _E), wvt, gmat,
      param.reshape(1, 2), b_val.reshape(1, 1))
    return acts, val.reshape(_B)


# speedup vs baseline: 8.2109x; 5.7699x over previous
import jax
import jax.numpy as jnp
from jax.experimental import pallas as pl
from jax.experimental.pallas import tpu as pltpu

_B, _N, _F, _E = 1024, 500, 64, 16
_TN = 20
_STEPS = _N // _TN
_AROWS = 2 * _TN


def _fused_body(xt_ref, wt_ref, bt_ref, wvb_ref, p_ref, bv_ref,
                actt_ref, val_ref, acc_ref):
    i = pl.program_id(0)

    @pl.when(i == 0)
    def _init():
        acc_ref[...] = jnp.zeros((_E, _B), jnp.float32)

    wt = wt_ref[...]
    bt = bt_ref[...]
    acc = acc_ref[...]
    for n in range(_TN):
        y = jnp.dot(wt, xt_ref[n], preferred_element_type=jnp.float32)
        z = jnp.maximum(y + bt, 0.0)
        acc = acc + z * wvb_ref[n * _E:(n + 1) * _E, :]
    acc_ref[...] = acc

    @pl.when(i == _STEPS - 1)
    def _finish():
        val_ref[...] = (jnp.sum(acc_ref[...], axis=0, keepdims=True)
                        + bv_ref[0, 0])

    a0 = jax.nn.sigmoid(p_ref[0, 0])
    a1 = jax.nn.sigmoid(p_ref[0, 1]) * 0.5
    r = jax.lax.broadcasted_iota(jnp.int32, (_AROWS, _B), 0)
    actt_ref[...] = jnp.where(r % 2 == 0, a0, a1)


def kernel(node_features_gen, W_embed, b_embed, param, W_val, b_val):
    xt = jnp.transpose(node_features_gen, (1, 2, 0))
    wt = W_embed.T

    actt, val = pl.pallas_call(
        _fused_body,
        grid=(_STEPS,),
        in_specs=[
            pl.BlockSpec((_TN, _F, _B), lambda i: (i, 0, 0)),
            pl.BlockSpec((_E, _F), lambda i: (0, 0)),
            pl.BlockSpec((_E, 1), lambda i: (0, 0)),
            pl.BlockSpec((_TN * _E, 1), lambda i: (i, 0)),
            pl.BlockSpec((1, 2), lambda i: (0, 0)),
            pl.BlockSpec((1, 1), lambda i: (0, 0)),
        ],
        out_specs=[
            pl.BlockSpec((_AROWS, _B), lambda i: (i, 0)),
            pl.BlockSpec((1, _B), lambda i: (0, 0)),
        ],
        out_shape=[
            jax.ShapeDtypeStruct((2 * _N, _B), jnp.float32),
            jax.ShapeDtypeStruct((1, _B), jnp.float32),
        ],
        scratch_shapes=[pltpu.VMEM((_E, _B), jnp.float32)],
    )(xt, wt, b_embed.reshape(_E, 1), W_val,
      param.reshape(1, 2), b_val.reshape(1, 1))
    return actt.T, val.reshape(_B)


# speedup vs baseline: 8.4030x; 1.0234x over previous
import jax
import jax.numpy as jnp
from jax.experimental import pallas as pl
from jax.experimental.pallas import tpu as pltpu

_B, _N, _F, _E = 1024, 500, 64, 16
_TN = 50
_STEPS = _N // _TN
_AROWS = 2 * _TN


def _fused_body(xt_ref, wt_ref, bt_ref, wvb_ref, p_ref, bv_ref,
                actt_ref, val_ref, acc_ref):
    i = pl.program_id(0)

    @pl.when(i == 0)
    def _init():
        acc_ref[...] = jnp.zeros((_E, _B), jnp.float32)

    wt = wt_ref[...]
    bt = bt_ref[...]
    acc = acc_ref[...]
    for n in range(_TN):
        y = jnp.dot(wt, xt_ref[n], preferred_element_type=jnp.float32)
        z = jnp.maximum(y + bt, 0.0)
        acc = acc + z * wvb_ref[n * _E:(n + 1) * _E, :]
    acc_ref[...] = acc

    @pl.when(i == _STEPS - 1)
    def _finish():
        val_ref[...] = (jnp.sum(acc_ref[...], axis=0, keepdims=True)
                        + bv_ref[0, 0])

    @pl.when(i == 0)
    def _actions():
        a0 = jax.nn.sigmoid(p_ref[0, 0])
        a1 = jax.nn.sigmoid(p_ref[0, 1]) * 0.5
        r = jax.lax.broadcasted_iota(jnp.int32, (2 * _N, _B), 0)
        actt_ref[...] = jnp.where(r % 2 == 0, a0, a1)


def kernel(node_features_gen, W_embed, b_embed, param, W_val, b_val):
    xt = jnp.transpose(node_features_gen, (1, 2, 0))
    wt = W_embed.T

    actt, val = pl.pallas_call(
        _fused_body,
        grid=(_STEPS,),
        in_specs=[
            pl.BlockSpec((_TN, _F, _B), lambda i: (i, 0, 0)),
            pl.BlockSpec((_E, _F), lambda i: (0, 0)),
            pl.BlockSpec((_E, 1), lambda i: (0, 0)),
            pl.BlockSpec((_TN * _E, 1), lambda i: (i, 0)),
            pl.BlockSpec((1, 2), lambda i: (0, 0)),
            pl.BlockSpec((1, 1), lambda i: (0, 0)),
        ],
        out_specs=[
            pl.BlockSpec((2 * _N, _B), lambda i: (0, 0)),
            pl.BlockSpec((1, _B), lambda i: (0, 0)),
        ],
        out_shape=[
            jax.ShapeDtypeStruct((2 * _N, _B), jnp.float32),
            jax.ShapeDtypeStruct((1, _B), jnp.float32),
        ],
        scratch_shapes=[pltpu.VMEM((_E, _B), jnp.float32)],
    )(xt, wt, b_embed.reshape(_E, 1), W_val,
      param.reshape(1, 2), b_val.reshape(1, 1))
    return actt.T, val.reshape(_B)


# speedup vs baseline: 8.6946x; 1.0347x over previous
import jax
import jax.numpy as jnp
from jax.experimental import pallas as pl
from jax.experimental.pallas import tpu as pltpu

_B, _N, _F, _E = 1024, 500, 64, 16
_TN = 25
_STEPS = _N // _TN
_AROWS = 2 * _TN


def _fused_body(xt_ref, wt_ref, bt_ref, wvb_ref, p_ref, bv_ref,
                actt_ref, val_ref, acc_ref):
    i = pl.program_id(0)

    @pl.when(i == 0)
    def _init():
        acc_ref[...] = jnp.zeros((_E, _B), jnp.float32)

    wt = wt_ref[...]
    bt = bt_ref[...]
    acc = acc_ref[...]
    for n in range(_TN):
        y = jnp.dot(wt, xt_ref[n], preferred_element_type=jnp.float32)
        z = jnp.maximum(y + bt, 0.0)
        acc = acc + z * wvb_ref[n * _E:(n + 1) * _E, :]
    acc_ref[...] = acc

    @pl.when(i == _STEPS - 1)
    def _finish():
        val_ref[...] = (jnp.sum(acc_ref[...], axis=0, keepdims=True)
                        + bv_ref[0, 0])

    @pl.when(i == 0)
    def _actions():
        a0 = jax.nn.sigmoid(p_ref[0, 0])
        a1 = jax.nn.sigmoid(p_ref[0, 1]) * 0.5
        r = jax.lax.broadcasted_iota(jnp.int32, (2 * _N, _B), 0)
        actt_ref[...] = jnp.where(r % 2 == 0, a0, a1)


def kernel(node_features_gen, W_embed, b_embed, param, W_val, b_val):
    xt = jnp.transpose(node_features_gen, (1, 2, 0))
    wt = W_embed.T

    actt, val = pl.pallas_call(
        _fused_body,
        grid=(_STEPS,),
        in_specs=[
            pl.BlockSpec((_TN, _F, _B), lambda i: (i, 0, 0)),
            pl.BlockSpec((_E, _F), lambda i: (0, 0)),
            pl.BlockSpec((_E, 1), lambda i: (0, 0)),
            pl.BlockSpec((_TN * _E, 1), lambda i: (i, 0)),
            pl.BlockSpec((1, 2), lambda i: (0, 0)),
            pl.BlockSpec((1, 1), lambda i: (0, 0)),
        ],
        out_specs=[
            pl.BlockSpec((2 * _N, _B), lambda i: (0, 0)),
            pl.BlockSpec((1, _B), lambda i: (0, 0)),
        ],
        out_shape=[
            jax.ShapeDtypeStruct((2 * _N, _B), jnp.float32),
            jax.ShapeDtypeStruct((1, _B), jnp.float32),
        ],
        scratch_shapes=[pltpu.VMEM((_E, _B), jnp.float32)],
    )(xt, wt, b_embed.reshape(_E, 1), W_val,
      param.reshape(1, 2), b_val.reshape(1, 1))
    return actt.T, val.reshape(_B)
